# baseline (speedup 1.0000x reference)
import functools

import jax
import jax.numpy as jnp
from jax import lax
from jax.experimental import pallas as pl
from jax.experimental.pallas import tpu as pltpu
from jax.experimental.pallas import tpu_sc as plsc

VOCAB = 1000000
EMBED_DIM = 32
BATCH = 16384
CTX = 20

SEG = 250112
NSTEP = SEG // 128
PROWS = 4 * SEG

NW = 32
ROWS_PER_W = BATCH // NW
CHUNK = 64
NCHUNK = ROWS_PER_W // CHUNK
IDX_PER_CHUNK = CHUNK * CTX
GATHERS = IDX_PER_CHUNK // 128


def _tc_pack_body(t0, t1, t2, t3, out_ref):
    parts = [t[...].T for t in (t0, t1, t2, t3)]
    out_ref[...] = jnp.concatenate(parts, axis=1)


@functools.lru_cache(maxsize=1)
def _tc_pack_call():
    in_specs = [
        pl.BlockSpec((EMBED_DIM, 128), functools.partial(lambda a, i: (0, NSTEP * a + i), a))
        for a in range(4)
    ]
    return pl.pallas_call(
        _tc_pack_body,
        grid=(NSTEP,),
        in_specs=in_specs,
        out_specs=pl.BlockSpec((128, 128), lambda i: (i, 0)),
        out_shape=jax.ShapeDtypeStruct((SEG, 128), jnp.float32),
    )


def _sc_cbow(idx_hbm, table_hbm, out_hbm, idx_v, rows_v, out_v, sem):
    nc = 2
    wid = lax.axis_index("s") * nc + lax.axis_index("c")
    base = wid * ROWS_PER_W
    inv_ctx = jnp.float32(1.0 / CTX)

    def chunk_body(c, _):
        pltpu.sync_copy(idx_hbm.at[wid, c], idx_v)
        copies = [
            pltpu.async_copy(
                table_hbm.at[idx_v.at[k]],
                rows_v.at[pl.ds(k * 128, 128)],
                sem,
            )
            for k in range(GATHERS)
        ]
        for cp in copies:
            cp.wait()

        def item_body(i, _):
            r0 = i * CTX
            acc0 = rows_v[r0, pl.ds(0, 16)]
            acc1 = rows_v[r0, pl.ds(16, 16)]
            for j in range(1, CTX):
                acc0 = acc0 + rows_v[r0 + j, pl.ds(0, 16)]
                acc1 = acc1 + rows_v[r0 + j, pl.ds(16, 16)]
            out_v[i, pl.ds(0, 16)] = acc0 * inv_ctx
            out_v[i, pl.ds(16, 16)] = acc1 * inv_ctx
            return 0

        lax.fori_loop(0, CHUNK, item_body, 0)
        pltpu.sync_copy(out_v, out_hbm.at[pl.ds(base + c * CHUNK, CHUNK)])
        return 0

    lax.fori_loop(0, NCHUNK, chunk_body, 0)


@functools.lru_cache(maxsize=1)
def _sc_call():
    return functools.partial(
        pl.kernel,
        mesh=plsc.VectorSubcoreMesh(core_axis_name="c", subcore_axis_name="s"),
        out_type=jax.ShapeDtypeStruct((BATCH, EMBED_DIM), jnp.float32),
        scratch_types=[
            pltpu.VMEM((GATHERS, 128), jnp.int32),
            pltpu.VMEM((IDX_PER_CHUNK, EMBED_DIM), jnp.float32),
            pltpu.VMEM((CHUNK, EMBED_DIM), jnp.float32),
            pltpu.SemaphoreType.DMA,
        ],
        compiler_params=pltpu.CompilerParams(use_tc_tiling_on_sc=False),
    )(_sc_cbow)


def kernel(inputs, table):
    idx = inputs.astype(jnp.int32)
    seg = idx // SEG
    g = idx * 4 - seg * (4 * SEG - 1)
    rows = g.reshape(NW, NCHUNK, GATHERS, 128)
    packed = _tc_pack_call()(table.T, table.T, table.T, table.T)
    packed_lin = packed.reshape(PROWS, EMBED_DIM)
    return _sc_call()(rows, packed_lin)


# speedup vs baseline: 1.9004x; 1.9004x over previous
import functools

import jax
import jax.numpy as jnp
from jax import lax
from jax.experimental import pallas as pl
from jax.experimental.pallas import tpu as pltpu
from jax.experimental.pallas import tpu_sc as plsc

VOCAB = 1000000
EMBED_DIM = 32
BATCH = 16384
CTX = 20

SEG = 250112
BLKN = 256
NSTEP = SEG // BLKN
PROWS = 4 * SEG

NW = 32
ROWS_PER_W = BATCH // NW
CHUNK = 64
NCHUNK = ROWS_PER_W // CHUNK
IDX_PER_CHUNK = CHUNK * CTX
GATHERS = IDX_PER_CHUNK // 128


def _tc_pack_body(t0, t1, t2, t3, out_ref):
    stacked = jnp.concatenate([t[...] for t in (t0, t1, t2, t3)], axis=0)
    out_ref[...] = stacked.T


@functools.lru_cache(maxsize=1)
def _tc_pack_call():
    last_blk = VOCAB // BLKN
    in_specs = [
        pl.BlockSpec(
            (EMBED_DIM, BLKN),
            functools.partial(lambda a, i: (0, jnp.minimum(NSTEP * a + i, last_blk)), a),
        )
        for a in range(4)
    ]
    return pl.pallas_call(
        _tc_pack_body,
        grid=(NSTEP,),
        in_specs=in_specs,
        out_specs=pl.BlockSpec((BLKN, 128), lambda i: (i, 0)),
        out_shape=jax.ShapeDtypeStruct((SEG, 128), jnp.float32),
    )


def _sc_cbow(idx_hbm, table_hbm, out_hbm, idx_v, rows_v, out_v, sem):
    nc = 2
    wid = lax.axis_index("s") * nc + lax.axis_index("c")
    base = wid * ROWS_PER_W
    inv_ctx = jnp.float32(1.0 / CTX)

    def chunk_body(c, _):
        pltpu.sync_copy(idx_hbm.at[wid, c], idx_v)
        copies = [
            pltpu.async_copy(
                table_hbm.at[idx_v.at[k]],
                rows_v.at[pl.ds(k * 128, 128)],
                sem,
            )
            for k in range(GATHERS)
        ]
        for cp in copies:
            cp.wait()

        def item_body(i, _):
            r0 = i * CTX
            acc0 = rows_v[r0, pl.ds(0, 16)]
            acc1 = rows_v[r0, pl.ds(16, 16)]
            for j in range(1, CTX):
                acc0 = acc0 + rows_v[r0 + j, pl.ds(0, 16)]
                acc1 = acc1 + rows_v[r0 + j, pl.ds(16, 16)]
            out_v[i, pl.ds(0, 16)] = acc0 * inv_ctx
            out_v[i, pl.ds(16, 16)] = acc1 * inv_ctx
            return 0

        lax.fori_loop(0, CHUNK, item_body, 0)
        pltpu.sync_copy(out_v, out_hbm.at[pl.ds(base + c * CHUNK, CHUNK)])
        return 0

    lax.fori_loop(0, NCHUNK, chunk_body, 0)


@functools.lru_cache(maxsize=1)
def _sc_call():
    return functools.partial(
        pl.kernel,
        mesh=plsc.VectorSubcoreMesh(core_axis_name="c", subcore_axis_name="s"),
        out_type=jax.ShapeDtypeStruct((BATCH, EMBED_DIM), jnp.float32),
        scratch_types=[
            pltpu.VMEM((GATHERS, 128), jnp.int32),
            pltpu.VMEM((IDX_PER_CHUNK, EMBED_DIM), jnp.float32),
            pltpu.VMEM((CHUNK, EMBED_DIM), jnp.float32),
            pltpu.SemaphoreType.DMA,
        ],
        compiler_params=pltpu.CompilerParams(use_tc_tiling_on_sc=False),
    )(_sc_cbow)


def kernel(inputs, table):
    idx = inputs.astype(jnp.int32)
    seg = idx // SEG
    g = idx * 4 - seg * (4 * SEG - 1)
    rows = g.reshape(NW, NCHUNK, GATHERS, 128)
    packed = _tc_pack_call()(table.T, table.T, table.T, table.T)
    packed_lin = packed.reshape(PROWS, EMBED_DIM)
    return _sc_call()(rows, packed_lin)


# speedup vs baseline: 4.4399x; 2.3363x over previous
import functools

import jax
import jax.numpy as jnp
from jax import lax
from jax.experimental import pallas as pl
from jax.experimental.pallas import tpu as pltpu
from jax.experimental.pallas import tpu_sc as plsc

VOCAB = 1000000
EMBED_DIM = 32
BATCH = 16384
CTX = 20

SEG = 250880
BLKN = 1024
NSTEP = SEG // BLKN
PROWS = 4 * SEG

NW = 32
ROWS_PER_W = BATCH // NW
CHUNK = 64
NCHUNK = ROWS_PER_W // CHUNK
IDX_PER_CHUNK = CHUNK * CTX
GATHERS = IDX_PER_CHUNK // 128


def _tc_pack_body(t0, t1, t2, t3, out_ref):
    stacked = jnp.concatenate([t[...] for t in (t0, t1, t2, t3)], axis=0)
    out_ref[...] = stacked.T


@functools.lru_cache(maxsize=1)
def _tc_pack_call():
    last_blk = VOCAB // BLKN
    in_specs = [
        pl.BlockSpec(
            (EMBED_DIM, BLKN),
            functools.partial(lambda a, i: (0, jnp.minimum(NSTEP * a + i, last_blk)), a),
        )
        for a in range(4)
    ]
    return pl.pallas_call(
        _tc_pack_body,
        grid=(NSTEP,),
        in_specs=in_specs,
        out_specs=pl.BlockSpec((BLKN, 128), lambda i: (i, 0)),
        out_shape=jax.ShapeDtypeStruct((SEG, 128), jnp.float32),
    )


def _sc_cbow(idx_hbm, table_hbm, out_hbm, idx_v, rows_v, out_v, sem):
    nc = 2
    wid = lax.axis_index("s") * nc + lax.axis_index("c")
    base = wid * ROWS_PER_W
    inv_ctx = jnp.float32(1.0 / CTX)

    def chunk_body(c, _):
        pltpu.sync_copy(idx_hbm.at[wid, c], idx_v)
        copies = [
            pltpu.async_copy(
                table_hbm.at[idx_v.at[k]],
                rows_v.at[pl.ds(k * 128, 128)],
                sem,
            )
            for k in range(GATHERS)
        ]
        for cp in copies:
            cp.wait()

        def item_body(i, _):
            r0 = i * CTX
            acc0 = rows_v[r0, pl.ds(0, 16)]
            acc1 = rows_v[r0, pl.ds(16, 16)]
            for j in range(1, CTX):
                acc0 = acc0 + rows_v[r0 + j, pl.ds(0, 16)]
                acc1 = acc1 + rows_v[r0 + j, pl.ds(16, 16)]
            out_v[i, pl.ds(0, 16)] = acc0 * inv_ctx
            out_v[i, pl.ds(16, 16)] = acc1 * inv_ctx
            return 0

        lax.fori_loop(0, CHUNK, item_body, 0)
        pltpu.sync_copy(out_v, out_hbm.at[pl.ds(base + c * CHUNK, CHUNK)])
        return 0

    lax.fori_loop(0, NCHUNK, chunk_body, 0)


@functools.lru_cache(maxsize=1)
def _sc_call():
    return functools.partial(
        pl.kernel,
        mesh=plsc.VectorSubcoreMesh(core_axis_name="c", subcore_axis_name="s"),
        out_type=jax.ShapeDtypeStruct((BATCH, EMBED_DIM), jnp.float32),
        scratch_types=[
            pltpu.VMEM((GATHERS, 128), jnp.int32),
            pltpu.VMEM((IDX_PER_CHUNK, EMBED_DIM), jnp.float32),
            pltpu.VMEM((CHUNK, EMBED_DIM), jnp.float32),
            pltpu.SemaphoreType.DMA,
        ],
        compiler_params=pltpu.CompilerParams(use_tc_tiling_on_sc=False),
    )(_sc_cbow)


def kernel(inputs, table):
    idx = inputs.astype(jnp.int32)
    seg = idx // SEG
    g = idx * 4 - seg * (4 * SEG - 1)
    rows = g.reshape(NW, NCHUNK, GATHERS, 128)
    packed = _tc_pack_call()(table.T, table.T, table.T, table.T)
    packed_lin = packed.reshape(PROWS, EMBED_DIM)
    return _sc_call()(rows, packed_lin)


# speedup vs baseline: 5.6112x; 1.2638x over previous
import functools

import jax
import jax.numpy as jnp
from jax import lax
from jax.experimental import pallas as pl
from jax.experimental.pallas import tpu as pltpu
from jax.experimental.pallas import tpu_sc as plsc

VOCAB = 1000000
EMBED_DIM = 32
BATCH = 16384
CTX = 20

SEG = 251904
BLKN = 2048
NSTEP = SEG // BLKN
PROWS = 4 * SEG

NW = 32
ROWS_PER_W = BATCH // NW
CHUNK = 64
NCHUNK = ROWS_PER_W // CHUNK
IDX_PER_CHUNK = CHUNK * CTX
GATHERS = IDX_PER_CHUNK // 128


def _tc_pack_body(t0, t1, t2, t3, out_ref):
    stacked = jnp.concatenate([t[...] for t in (t0, t1, t2, t3)], axis=0)
    out_ref[...] = stacked.T


@functools.lru_cache(maxsize=1)
def _tc_pack_call():
    last_blk = VOCAB // BLKN
    in_specs = [
        pl.BlockSpec(
            (EMBED_DIM, BLKN),
            functools.partial(lambda a, i: (0, jnp.minimum(NSTEP * a + i, last_blk)), a),
        )
        for a in range(4)
    ]
    return pl.pallas_call(
        _tc_pack_body,
        grid=(NSTEP,),
        in_specs=in_specs,
        out_specs=pl.BlockSpec((BLKN, 128), lambda i: (i, 0)),
        out_shape=jax.ShapeDtypeStruct((SEG, 128), jnp.float32),
    )


def _sc_cbow(idx_hbm, table_hbm, out_hbm, idx_v, rows_v, out_v, sem):
    nc = 2
    wid = lax.axis_index("s") * nc + lax.axis_index("c")
    base = wid * ROWS_PER_W
    inv_ctx = jnp.float32(1.0 / CTX)

    def chunk_body(c, _):
        pltpu.sync_copy(idx_hbm.at[wid, c], idx_v)
        copies = [
            pltpu.async_copy(
                table_hbm.at[idx_v.at[k]],
                rows_v.at[pl.ds(k * 128, 128)],
                sem,
            )
            for k in range(GATHERS)
        ]
        for cp in copies:
            cp.wait()

        def item_body(i, _):
            r0 = i * CTX
            acc0 = rows_v[r0, pl.ds(0, 16)]
            acc1 = rows_v[r0, pl.ds(16, 16)]
            for j in range(1, CTX):
                acc0 = acc0 + rows_v[r0 + j, pl.ds(0, 16)]
                acc1 = acc1 + rows_v[r0 + j, pl.ds(16, 16)]
            out_v[i, pl.ds(0, 16)] = acc0 * inv_ctx
            out_v[i, pl.ds(16, 16)] = acc1 * inv_ctx
            return 0

        lax.fori_loop(0, CHUNK, item_body, 0)
        pltpu.sync_copy(out_v, out_hbm.at[pl.ds(base + c * CHUNK, CHUNK)])
        return 0

    lax.fori_loop(0, NCHUNK, chunk_body, 0)


@functools.lru_cache(maxsize=1)
def _sc_call():
    return functools.partial(
        pl.kernel,
        mesh=plsc.VectorSubcoreMesh(core_axis_name="c", subcore_axis_name="s"),
        out_type=jax.ShapeDtypeStruct((BATCH, EMBED_DIM), jnp.float32),
        scratch_types=[
            pltpu.VMEM((GATHERS, 128), jnp.int32),
            pltpu.VMEM((IDX_PER_CHUNK, EMBED_DIM), jnp.float32),
            pltpu.VMEM((CHUNK, EMBED_DIM), jnp.float32),
            pltpu.SemaphoreType.DMA,
        ],
        compiler_params=pltpu.CompilerParams(use_tc_tiling_on_sc=False),
    )(_sc_cbow)


def kernel(inputs, table):
    idx = inputs.astype(jnp.int32)
    seg = idx // SEG
    g = idx * 4 - seg * (4 * SEG - 1)
    rows = g.reshape(NW, NCHUNK, GATHERS, 128)
    packed = _tc_pack_call()(table.T, table.T, table.T, table.T)
    packed_lin = packed.reshape(PROWS, EMBED_DIM)
    return _sc_call()(rows, packed_lin)


# speedup vs baseline: 6.6902x; 1.1923x over previous
import functools

import jax
import jax.numpy as jnp
from jax import lax
from jax.experimental import pallas as pl
from jax.experimental.pallas import tpu as pltpu
from jax.experimental.pallas import tpu_sc as plsc

VOCAB = 1000000
EMBED_DIM = 32
BATCH = 16384
CTX = 20

SEG = 253952
BLKN = 4096
NSTEP = SEG // BLKN
PROWS = 4 * SEG

NW = 32
ROWS_PER_W = BATCH // NW
CHUNK = 64
NCHUNK = ROWS_PER_W // CHUNK
IDX_PER_CHUNK = CHUNK * CTX
GATHERS = IDX_PER_CHUNK // 128


def _tc_pack_body(t0, t1, t2, t3, out_ref):
    stacked = jnp.concatenate([t[...] for t in (t0, t1, t2, t3)], axis=0)
    out_ref[...] = stacked.T


@functools.lru_cache(maxsize=1)
def _tc_pack_call():
    last_blk = VOCAB // BLKN
    in_specs = [
        pl.BlockSpec(
            (EMBED_DIM, BLKN),
            functools.partial(lambda a, i: (0, jnp.minimum(NSTEP * a + i, last_blk)), a),
        )
        for a in range(4)
    ]
    return pl.pallas_call(
        _tc_pack_body,
        grid=(NSTEP,),
        in_specs=in_specs,
        out_specs=pl.BlockSpec((BLKN, 128), lambda i: (i, 0)),
        out_shape=jax.ShapeDtypeStruct((SEG, 128), jnp.float32),
    )


def _sc_cbow(idx_hbm, table_hbm, out_hbm, idx_v, rows_v, out_v, sem):
    nc = 2
    wid = lax.axis_index("s") * nc + lax.axis_index("c")
    base = wid * ROWS_PER_W
    inv_ctx = jnp.float32(1.0 / CTX)

    def chunk_body(c, _):
        pltpu.sync_copy(idx_hbm.at[wid, c], idx_v)
        copies = [
            pltpu.async_copy(
                table_hbm.at[idx_v.at[k]],
                rows_v.at[pl.ds(k * 128, 128)],
                sem,
            )
            for k in range(GATHERS)
        ]
        for cp in copies:
            cp.wait()

        def item_body(i, _):
            r0 = i * CTX
            acc0 = rows_v[r0, pl.ds(0, 16)]
            acc1 = rows_v[r0, pl.ds(16, 16)]
            for j in range(1, CTX):
                acc0 = acc0 + rows_v[r0 + j, pl.ds(0, 16)]
                acc1 = acc1 + rows_v[r0 + j, pl.ds(16, 16)]
            out_v[i, pl.ds(0, 16)] = acc0 * inv_ctx
            out_v[i, pl.ds(16, 16)] = acc1 * inv_ctx
            return 0

        lax.fori_loop(0, CHUNK, item_body, 0)
        pltpu.sync_copy(out_v, out_hbm.at[pl.ds(base + c * CHUNK, CHUNK)])
        return 0

    lax.fori_loop(0, NCHUNK, chunk_body, 0)


@functools.lru_cache(maxsize=1)
def _sc_call():
    return functools.partial(
        pl.kernel,
        mesh=plsc.VectorSubcoreMesh(core_axis_name="c", subcore_axis_name="s"),
        out_type=jax.ShapeDtypeStruct((BATCH, EMBED_DIM), jnp.float32),
        scratch_types=[
            pltpu.VMEM((GATHERS, 128), jnp.int32),
            pltpu.VMEM((IDX_PER_CHUNK, EMBED_DIM), jnp.float32),
            pltpu.VMEM((CHUNK, EMBED_DIM), jnp.float32),
            pltpu.SemaphoreType.DMA,
        ],
        compiler_params=pltpu.CompilerParams(use_tc_tiling_on_sc=False),
    )(_sc_cbow)


def kernel(inputs, table):
    idx = inputs.astype(jnp.int32)
    seg = idx // SEG
    g = idx * 4 - seg * (4 * SEG - 1)
    rows = g.reshape(NW, NCHUNK, GATHERS, 128)
    packed = _tc_pack_call()(table.T, table.T, table.T, table.T)
    packed_lin = packed.reshape(PROWS, EMBED_DIM)
    return _sc_call()(rows, packed_lin)


# speedup vs baseline: 7.2489x; 1.0835x over previous
import functools

import jax
import jax.numpy as jnp
from jax import lax
from jax.experimental import pallas as pl
from jax.experimental.pallas import tpu as pltpu
from jax.experimental.pallas import tpu_sc as plsc

VOCAB = 1000000
EMBED_DIM = 32
BATCH = 16384
CTX = 20

SEG = 253952
BLKN = 8192
NSTEP = SEG // BLKN
PROWS = 4 * SEG

NW = 32
ROWS_PER_W = BATCH // NW
CHUNK = 64
NCHUNK = ROWS_PER_W // CHUNK
IDX_PER_CHUNK = CHUNK * CTX
GATHERS = IDX_PER_CHUNK // 128


def _tc_pack_body(t0, t1, t2, t3, out_ref):
    stacked = jnp.concatenate([t[...] for t in (t0, t1, t2, t3)], axis=0)
    out_ref[...] = stacked.T


@functools.lru_cache(maxsize=1)
def _tc_pack_call():
    last_blk = VOCAB // BLKN
    in_specs = [
        pl.BlockSpec(
            (EMBED_DIM, BLKN),
            functools.partial(lambda a, i: (0, jnp.minimum(NSTEP * a + i, last_blk)), a),
        )
        for a in range(4)
    ]
    return pl.pallas_call(
        _tc_pack_body,
        grid=(NSTEP,),
        in_specs=in_specs,
        out_specs=pl.BlockSpec((BLKN, 128), lambda i: (i, 0)),
        out_shape=jax.ShapeDtypeStruct((SEG, 128), jnp.float32),
    )


def _sc_cbow(idx_hbm, table_hbm, out_hbm, idx_v, rows_v, out_v, sem):
    nc = 2
    wid = lax.axis_index("s") * nc + lax.axis_index("c")
    base = wid * ROWS_PER_W
    inv_ctx = jnp.float32(1.0 / CTX)

    def chunk_body(c, _):
        pltpu.sync_copy(idx_hbm.at[wid, c], idx_v)
        copies = [
            pltpu.async_copy(
                table_hbm.at[idx_v.at[k]],
                rows_v.at[pl.ds(k * 128, 128)],
                sem,
            )
            for k in range(GATHERS)
        ]
        for cp in copies:
            cp.wait()

        def item_body(i, _):
            r0 = i * CTX
            acc0 = rows_v[r0, pl.ds(0, 16)]
            acc1 = rows_v[r0, pl.ds(16, 16)]
            for j in range(1, CTX):
                acc0 = acc0 + rows_v[r0 + j, pl.ds(0, 16)]
                acc1 = acc1 + rows_v[r0 + j, pl.ds(16, 16)]
            out_v[i, pl.ds(0, 16)] = acc0 * inv_ctx
            out_v[i, pl.ds(16, 16)] = acc1 * inv_ctx
            return 0

        lax.fori_loop(0, CHUNK, item_body, 0)
        pltpu.sync_copy(out_v, out_hbm.at[pl.ds(base + c * CHUNK, CHUNK)])
        return 0

    lax.fori_loop(0, NCHUNK, chunk_body, 0)


@functools.lru_cache(maxsize=1)
def _sc_call():
    return functools.partial(
        pl.kernel,
        mesh=plsc.VectorSubcoreMesh(core_axis_name="c", subcore_axis_name="s"),
        out_type=jax.ShapeDtypeStruct((BATCH, EMBED_DIM), jnp.float32),
        scratch_types=[
            pltpu.VMEM((GATHERS, 128), jnp.int32),
            pltpu.VMEM((IDX_PER_CHUNK, EMBED_DIM), jnp.float32),
            pltpu.VMEM((CHUNK, EMBED_DIM), jnp.float32),
            pltpu.SemaphoreType.DMA,
        ],
        compiler_params=pltpu.CompilerParams(use_tc_tiling_on_sc=False),
    )(_sc_cbow)


def kernel(inputs, table):
    idx = inputs.astype(jnp.int32)
    seg = idx // SEG
    g = idx * 4 - seg * (4 * SEG - 1)
    rows = g.reshape(NW, NCHUNK, GATHERS, 128)
    packed = _tc_pack_call()(table.T, table.T, table.T, table.T)
    packed_lin = packed.reshape(PROWS, EMBED_DIM)
    return _sc_call()(rows, packed_lin)


# speedup vs baseline: 7.3258x; 1.0106x over previous
import functools

import jax
import jax.numpy as jnp
from jax import lax
from jax.experimental import pallas as pl
from jax.experimental.pallas import tpu as pltpu
from jax.experimental.pallas import tpu_sc as plsc

VOCAB = 1000000
EMBED_DIM = 32
BATCH = 16384
CTX = 20

SEG = 262144
BLKN = 16384
NSTEP = SEG // BLKN
PROWS = 4 * SEG

NW = 32
ROWS_PER_W = BATCH // NW
CHUNK = 64
NCHUNK = ROWS_PER_W // CHUNK
IDX_PER_CHUNK = CHUNK * CTX
GATHERS = IDX_PER_CHUNK // 128


def _tc_pack_body(t0, t1, t2, t3, out_ref):
    stacked = jnp.concatenate([t[...] for t in (t0, t1, t2, t3)], axis=0)
    out_ref[...] = stacked.T


@functools.lru_cache(maxsize=1)
def _tc_pack_call():
    last_blk = VOCAB // BLKN
    in_specs = [
        pl.BlockSpec(
            (EMBED_DIM, BLKN),
            functools.partial(lambda a, i: (0, jnp.minimum(NSTEP * a + i, last_blk)), a),
        )
        for a in range(4)
    ]
    return pl.pallas_call(
        _tc_pack_body,
        grid=(NSTEP,),
        in_specs=in_specs,
        out_specs=pl.BlockSpec((BLKN, 128), lambda i: (i, 0)),
        out_shape=jax.ShapeDtypeStruct((SEG, 128), jnp.float32),
    )


def _sc_cbow(idx_hbm, table_hbm, out_hbm, idx_v, rows_v, out_v, sem):
    nc = 2
    wid = lax.axis_index("s") * nc + lax.axis_index("c")
    base = wid * ROWS_PER_W
    inv_ctx = jnp.float32(1.0 / CTX)

    def chunk_body(c, _):
        pltpu.sync_copy(idx_hbm.at[wid, c], idx_v)
        copies = [
            pltpu.async_copy(
                table_hbm.at[idx_v.at[k]],
                rows_v.at[pl.ds(k * 128, 128)],
                sem,
            )
            for k in range(GATHERS)
        ]
        for cp in copies:
            cp.wait()

        def item_body(i, _):
            r0 = i * CTX
            acc0 = rows_v[r0, pl.ds(0, 16)]
            acc1 = rows_v[r0, pl.ds(16, 16)]
            for j in range(1, CTX):
                acc0 = acc0 + rows_v[r0 + j, pl.ds(0, 16)]
                acc1 = acc1 + rows_v[r0 + j, pl.ds(16, 16)]
            out_v[i, pl.ds(0, 16)] = acc0 * inv_ctx
            out_v[i, pl.ds(16, 16)] = acc1 * inv_ctx
            return 0

        lax.fori_loop(0, CHUNK, item_body, 0)
        pltpu.sync_copy(out_v, out_hbm.at[pl.ds(base + c * CHUNK, CHUNK)])
        return 0

    lax.fori_loop(0, NCHUNK, chunk_body, 0)


@functools.lru_cache(maxsize=1)
def _sc_call():
    return functools.partial(
        pl.kernel,
        mesh=plsc.VectorSubcoreMesh(core_axis_name="c", subcore_axis_name="s"),
        out_type=jax.ShapeDtypeStruct((BATCH, EMBED_DIM), jnp.float32),
        scratch_types=[
            pltpu.VMEM((GATHERS, 128), jnp.int32),
            pltpu.VMEM((IDX_PER_CHUNK, EMBED_DIM), jnp.float32),
            pltpu.VMEM((CHUNK, EMBED_DIM), jnp.float32),
            pltpu.SemaphoreType.DMA,
        ],
        compiler_params=pltpu.CompilerParams(use_tc_tiling_on_sc=False),
    )(_sc_cbow)


def kernel(inputs, table):
    idx = inputs.astype(jnp.int32)
    seg = idx // SEG
    g = idx * 4 - seg * (4 * SEG - 1)
    rows = g.reshape(NW, NCHUNK, GATHERS, 128)
    packed = _tc_pack_call()(table.T, table.T, table.T, table.T)
    packed_lin = packed.reshape(PROWS, EMBED_DIM)
    return _sc_call()(rows, packed_lin)
